# Initial kernel scaffold; baseline (speedup 1.0000x reference)
import jax
import jax.numpy as jnp
from jax.experimental import pallas as pl


def kernel(image_features, key_features, cls_attn, similarity):
    raise NotImplementedError("write your pallas kernel here")


# speedup vs baseline: 6.2348x; 6.2348x over previous
import jax
import jax.numpy as jnp
from jax.experimental import pallas as pl

N = 1024
C = 1024
KV = 128
KT = 128
KSEL = KV + KT
K2 = int((N - KSEL) * 0.25)
NOUT = KSEL + K2
SCALE = C ** -0.5


def _body(img_ref, key_ref, ar_ref, ac_ref, sr_ref, sc_ref, out_ref):
    f32 = jnp.float32
    ar = ar_ref[...]
    ac = ac_ref[...]
    sr = sr_ref[...]
    sc = sc_ref[...]

    ioj = jax.lax.broadcasted_iota(jnp.int32, (N, N), 0)
    ioi = jax.lax.broadcasted_iota(jnp.int32, (N, N), 1)

    def rank_row(vc, vr):
        m = (vc > vr) | ((vc == vr) & (ioj < ioi))
        return jnp.sum(m.astype(f32), axis=0, keepdims=True)

    def rank_col(vc, vr):
        m = (vr > vc) | ((vr == vc) & (ioi < ioj))
        return jnp.sum(m.astype(f32), axis=1, keepdims=True)

    rv_r = rank_row(ac, ar)
    rt_r = rank_row(sc, sr)
    rv_c = rank_col(ac, ar)
    rt_c = rank_col(sc, sr)
    sel_r = ((rv_r < KV) | (rt_r < KT)).astype(f32)
    sel_c = ((rv_c < KV) | (rt_c < KT)).astype(f32)
    k1_r = sel_r * 1e6 + ar
    k1_c = sel_c * 1e6 + ac
    rs_r = rank_row(k1_c, k1_r)
    rs_c = rank_col(k1_c, k1_r)
    a_r = rs_r < KSEL
    a_c = rs_c < KSEL

    m2 = (~a_c) & ((ac > ar) | ((ac == ar) & (ioj < ioi)))
    r2_r = jnp.sum(m2.astype(f32), axis=0, keepdims=True)
    b_r = (~a_r) & (r2_r < K2)
    cmask = (~a_r) & (~b_r)

    row_of = jnp.where(a_r, rs_r, jnp.where(b_r, KSEL + r2_r, 2.0 * N))
    io_out = jax.lax.broadcasted_iota(jnp.int32, (NOUT, N), 0)
    p = (io_out == row_of.astype(jnp.int32)).astype(f32)

    img = img_ref[...]
    key = key_ref[...]
    x = jax.lax.dot_general(p, img, (((1,), (0,)), ((), ())),
                            preferred_element_type=f32)
    kb = jax.lax.dot_general(p[KSEL:, :], key, (((1,), (0,)), ((), ())),
                             preferred_element_type=f32)

    nb = jnp.sqrt(jnp.sum(kb * kb, axis=1, keepdims=True))
    kbn = kb / jnp.maximum(nb, 1e-12)
    nk = jnp.sqrt(jnp.sum(key * key, axis=1, keepdims=True))
    kn = key / jnp.maximum(nk, 1e-12)
    cos = jax.lax.dot_general(kbn, kn, (((1,), (1,)), ((), ())),
                              preferred_element_type=f32) * SCALE

    neg = jnp.float32(-jnp.inf)
    logits = jnp.where(cmask, cos, neg)
    mx = jnp.max(logits, axis=1, keepdims=True)
    e = jnp.exp(logits - mx)
    w = e / jnp.sum(e, axis=1, keepdims=True)

    t = jnp.where(cmask, 50.0 * sr, neg)
    tm = jnp.max(t, axis=1, keepdims=True)
    te = jnp.exp(t - tm)
    sm = te / jnp.sum(te, axis=1, keepdims=True)
    score = ar * sm

    ws = w * score
    fused = jax.lax.dot_general(ws, img, (((1,), (0,)), ((), ())),
                                preferred_element_type=f32)

    out_ref[0:KSEL, :] = x[0:KSEL, :]
    out_ref[KSEL:NOUT, :] = x[KSEL:NOUT, :] + fused


def kernel(image_features, key_features, cls_attn, similarity):
    img = image_features[0]
    key = key_features[0]
    ar = cls_attn
    ac = cls_attn.reshape(N, 1)
    sr = similarity
    sc = similarity.reshape(N, 1)
    out = pl.pallas_call(
        _body,
        out_shape=jax.ShapeDtypeStruct((NOUT, C), jnp.float32),
    )(img, key, ar, ac, sr, sc)
    return out[None]
